# baseline (speedup 1.0000x reference)
import functools

import jax
import jax.numpy as jnp
from jax import lax
from jax.experimental import pallas as pl
from jax.experimental.pallas import tpu as pltpu
from jax.experimental.pallas import tpu_sc as plsc

DIMK = 64
JC = 2
IB = 256
L = 16


def _sc_body(xt_hbm, z_hbm, out_hbm,
             ixa0, ixa1, ixb0, ixb1, oh0, oh1,
             isa0, isa1, isb0, isb1, sem0, sem1):
    jpc = xt_hbm.shape[0] // 2
    jbase = lax.axis_index("c") * jpc
    i0 = lax.axis_index("s") * IB
    n_chunks = jpc // JC
    jmax = (n_chunks - 1) * JC

    ones = jnp.full((L,), 1.0, jnp.float32)
    zeros = jnp.zeros((L,), jnp.float32)
    lane = lax.iota(jnp.int32, L)

    def scatter_vals(oh, ix, vals):
        for jj in range(JC):
            for b in range(IB // L):
                xv = ix[jj, pl.ds(L * b, L)]
                plsc.store_scatter(oh.at[jj], [xv, lane + (L * b)], vals)

    def load_ix(g, ix, isem):
        j0 = jbase + jnp.minimum(g * JC, jmax)
        pltpu.async_copy(xt_hbm.at[pl.ds(j0, JC), pl.ds(i0, IB)], ix, isem)

    def run_chunk(g, ix, isem, ixo, isemo, oh, sem, first):
        dst = out_hbm.at[pl.ds(jbase + g * JC, JC), :, pl.ds(i0, IB)]
        if not first:
            pltpu.make_async_copy(oh, dst, sem).wait()
            scatter_vals(oh, ixo, zeros)
            load_ix(g + 2, ixo, isemo)
        pltpu.make_async_copy(xt_hbm.at[pl.ds(0, JC), pl.ds(i0, IB)],
                              ix, isem).wait()
        scatter_vals(oh, ix, ones)
        pltpu.async_copy(oh, dst, sem)

    load_ix(0, ixa0, isa0)
    load_ix(1, ixa1, isa1)
    load_ix(2, ixb0, isb0)
    load_ix(3, ixb1, isb1)
    pltpu.sync_copy(z_hbm, oh0)
    pltpu.sync_copy(z_hbm, oh1)
    run_chunk(0, ixa0, isa0, None, None, oh0, sem0, first=True)
    run_chunk(1, ixa1, isa1, None, None, oh1, sem1, first=True)

    def quad(m, carry):
        g = 4 * m + 2
        run_chunk(g, ixb0, isb0, ixa0, isa0, oh0, sem0, first=False)
        run_chunk(g + 1, ixb1, isb1, ixa1, isa1, oh1, sem1, first=False)
        run_chunk(g + 2, ixa0, isa0, ixb0, isb0, oh0, sem0, first=False)
        run_chunk(g + 3, ixa1, isa1, ixb1, isb1, oh1, sem1, first=False)
        return carry

    lax.fori_loop(0, (n_chunks - 2) // 4, quad, 0, unroll=False)

    dst0 = out_hbm.at[pl.ds(0, JC), :, pl.ds(i0, IB)]
    src0 = xt_hbm.at[pl.ds(0, JC), pl.ds(i0, IB)]
    pltpu.make_async_copy(oh0, dst0, sem0).wait()
    pltpu.make_async_copy(oh1, dst0, sem1).wait()
    pltpu.make_async_copy(src0, ixb0, isb0).wait()
    pltpu.make_async_copy(src0, ixb1, isb1).wait()


def kernel(x, I):
    nj = x.shape[1]
    ni = x.shape[0]
    xt = jnp.transpose(x)
    z = jnp.zeros((JC, DIMK, IB), jnp.float32)
    mesh = plsc.VectorSubcoreMesh(core_axis_name="c", subcore_axis_name="s")
    f = functools.partial(
        pl.kernel,
        mesh=mesh,
        out_type=jax.ShapeDtypeStruct((nj, DIMK, ni), jnp.float32),
        scratch_types=[
            pltpu.VMEM((JC, IB), jnp.int32),
            pltpu.VMEM((JC, IB), jnp.int32),
            pltpu.VMEM((JC, IB), jnp.int32),
            pltpu.VMEM((JC, IB), jnp.int32),
            pltpu.VMEM((JC, DIMK, IB), jnp.float32),
            pltpu.VMEM((JC, DIMK, IB), jnp.float32),
            pltpu.SemaphoreType.DMA,
            pltpu.SemaphoreType.DMA,
            pltpu.SemaphoreType.DMA,
            pltpu.SemaphoreType.DMA,
            pltpu.SemaphoreType.DMA,
            pltpu.SemaphoreType.DMA,
        ],
        compiler_params=pltpu.CompilerParams(
            needs_layout_passes=False,
        ),
    )(_sc_body)
    outT = f(xt, z)
    return jnp.transpose(outT, (2, 0, 1))


# speedup vs baseline: 1.0136x; 1.0136x over previous
import functools

import jax
import jax.numpy as jnp
from jax import lax
from jax.experimental import pallas as pl
from jax.experimental.pallas import tpu as pltpu
from jax.experimental.pallas import tpu_sc as plsc

DIMK = 64
JC = 4
IB = 128
L = 16


def _sc_body(xt_hbm, z_hbm, out_hbm,
             ixa0, ixa1, ixb0, ixb1, oh0, oh1,
             isa0, isa1, isb0, isb1, sem0, sem1):
    nc = 2
    wid = lax.axis_index("s") * nc + lax.axis_index("c")
    i0 = wid * IB
    n_chunks = xt_hbm.shape[0] // JC
    jmax = (n_chunks - 1) * JC

    ones = jnp.full((L,), 1.0, jnp.float32)
    zeros = jnp.zeros((L,), jnp.float32)
    lane = lax.iota(jnp.int32, L)

    def scatter_vals(oh, ix, vals):
        for jj in range(JC):
            for b in range(IB // L):
                xv = ix[jj, pl.ds(L * b, L)]
                plsc.store_scatter(oh.at[jj], [xv, lane + (L * b)], vals)

    def load_ix(g, ix, isem):
        j0 = jnp.minimum(g * JC, jmax)
        pltpu.async_copy(xt_hbm.at[pl.ds(j0, JC), pl.ds(i0, IB)], ix, isem)

    def run_chunk(g, ix, isem, ixo, isemo, oh, sem, first):
        dst = out_hbm.at[pl.ds(g * JC, JC), :, pl.ds(i0, IB)]
        if not first:
            pltpu.make_async_copy(oh, dst, sem).wait()
            scatter_vals(oh, ixo, zeros)
            load_ix(g + 2, ixo, isemo)
        pltpu.make_async_copy(xt_hbm.at[pl.ds(0, JC), pl.ds(i0, IB)],
                              ix, isem).wait()
        scatter_vals(oh, ix, ones)
        pltpu.async_copy(oh, dst, sem)

    load_ix(0, ixa0, isa0)
    load_ix(1, ixa1, isa1)
    load_ix(2, ixb0, isb0)
    load_ix(3, ixb1, isb1)
    pltpu.async_copy(z_hbm, oh0, sem0)
    pltpu.async_copy(z_hbm, oh1, sem1)
    pltpu.make_async_copy(z_hbm, oh0, sem0).wait()
    pltpu.make_async_copy(z_hbm, oh1, sem1).wait()
    run_chunk(0, ixa0, isa0, None, None, oh0, sem0, first=True)
    run_chunk(1, ixa1, isa1, None, None, oh1, sem1, first=True)

    def quad(m, carry):
        g = 4 * m + 2
        run_chunk(g, ixb0, isb0, ixa0, isa0, oh0, sem0, first=False)
        run_chunk(g + 1, ixb1, isb1, ixa1, isa1, oh1, sem1, first=False)
        run_chunk(g + 2, ixa0, isa0, ixb0, isb0, oh0, sem0, first=False)
        run_chunk(g + 3, ixa1, isa1, ixb1, isb1, oh1, sem1, first=False)
        return carry

    lax.fori_loop(0, (n_chunks - 2) // 4, quad, 0, unroll=False)

    dst0 = out_hbm.at[pl.ds(0, JC), :, pl.ds(i0, IB)]
    src0 = xt_hbm.at[pl.ds(0, JC), pl.ds(i0, IB)]
    pltpu.make_async_copy(oh0, dst0, sem0).wait()
    pltpu.make_async_copy(oh1, dst0, sem1).wait()
    pltpu.make_async_copy(src0, ixb0, isb0).wait()
    pltpu.make_async_copy(src0, ixb1, isb1).wait()


def kernel(x, I):
    nj = x.shape[1]
    ni = x.shape[0]
    xt = jnp.transpose(x)
    z = jnp.zeros((JC, DIMK, IB), jnp.float32)
    mesh = plsc.VectorSubcoreMesh(core_axis_name="c", subcore_axis_name="s")
    f = functools.partial(
        pl.kernel,
        mesh=mesh,
        out_type=jax.ShapeDtypeStruct((nj, DIMK, ni), jnp.float32),
        scratch_types=[
            pltpu.VMEM((JC, IB), jnp.int32),
            pltpu.VMEM((JC, IB), jnp.int32),
            pltpu.VMEM((JC, IB), jnp.int32),
            pltpu.VMEM((JC, IB), jnp.int32),
            pltpu.VMEM((JC, DIMK, IB), jnp.float32),
            pltpu.VMEM((JC, DIMK, IB), jnp.float32),
            pltpu.SemaphoreType.DMA,
            pltpu.SemaphoreType.DMA,
            pltpu.SemaphoreType.DMA,
            pltpu.SemaphoreType.DMA,
            pltpu.SemaphoreType.DMA,
            pltpu.SemaphoreType.DMA,
        ],
        compiler_params=pltpu.CompilerParams(
            needs_layout_passes=False,
        ),
    )(_sc_body)
    outT = f(xt, z)
    return jnp.transpose(outT, (2, 0, 1))
